# Initial kernel scaffold; baseline (speedup 1.0000x reference)
import jax
import jax.numpy as jnp
from jax.experimental import pallas as pl


def kernel(x, emb_table, lin_table, bias, p, W1, b1, W2):
    raise NotImplementedError("write your pallas kernel here")


# speedup vs baseline: 2.8024x; 2.8024x over previous
import functools

import numpy as np
import jax
import jax.numpy as jnp
from jax import lax
from jax.experimental import pallas as pl
from jax.experimental.pallas import tpu as pltpu
from jax.experimental.pallas import tpu_sc as plsc

_B = 4096
_F = 26
_D = 32
_P = _F * (_F - 1) // 2

_In, _Jn = np.triu_indices(_F, k=1)
_OHI = np.zeros((_F, _P), np.float32)
_OHJ = np.zeros((_F, _P), np.float32)
_OHI[_In, np.arange(_P)] = 1.0
_OHJ[_Jn, np.arange(_P)] = 1.0

_NC = 2
_NS = 16
_NW = _NC * _NS
_TOT = _B * _F
_BPW = _TOT // _NW


def _sc_body(tab_hbm, lin_hbm, idx_hbm, rows_hbm, linr_hbm,
             idx_v, rows_v, linr_v, sem_e, sem_l):
    wid = lax.axis_index("s") * _NC + lax.axis_index("c")
    base = wid * _BPW
    pltpu.sync_copy(idx_hbm.at[pl.ds(base, _BPW)], idx_v)
    cp_e = pltpu.async_copy(tab_hbm.at[idx_v], rows_v, sem_e)
    cp_l = pltpu.async_copy(lin_hbm.at[idx_v], linr_v, sem_l)
    cp_e.wait()
    cp_l.wait()
    pltpu.sync_copy(rows_v, rows_hbm.at[pl.ds(base, _BPW)])
    pltpu.sync_copy(linr_v, linr_hbm.at[pl.ds(base, _BPW)])


@functools.cache
def _sc_gather_fn():
    return functools.partial(
        pl.kernel,
        out_type=[
            jax.ShapeDtypeStruct((_TOT, _D), jnp.float32),
            jax.ShapeDtypeStruct((_TOT,), jnp.float32),
        ],
        mesh=plsc.VectorSubcoreMesh(core_axis_name="c", subcore_axis_name="s"),
        compiler_params=pltpu.CompilerParams(use_tc_tiling_on_sc=False),
        scratch_types=[
            pltpu.VMEM((_BPW,), jnp.int32),
            pltpu.VMEM((_BPW, _D), jnp.float32),
            pltpu.VMEM((_BPW,), jnp.float32),
            pltpu.SemaphoreType.DMA,
            pltpu.SemaphoreType.DMA,
        ],
    )(_sc_body)


_BLK = 128


def _tc_body(e2_ref, linr_ref, ohi_ref, ohj_ref, w15_ref, b1_ref, w2_ref,
             bias_ref, o_ref):
    e2 = e2_ref[...].reshape(_D * _BLK, _F)
    ei = jnp.dot(e2, ohi_ref[...], preferred_element_type=jnp.float32)
    ej = jnp.dot(e2, ohj_ref[...], preferred_element_type=jnp.float32)
    z3 = (ei * ej).reshape(_D, _BLK, _P)
    accs = [jnp.zeros((_BLK, _P), jnp.float32) for _ in range(5)]
    for d in range(_D):
        t = z3[d]
        for k in range(5):
            accs[k] = accs[k] + t * w15_ref[k, d]
    logits = sum(
        jnp.maximum(accs[k] + b1_ref[0, k], 0.0) * w2_ref[0, k]
        for k in range(4)
    )
    s = accs[4]
    m = jnp.max(logits, axis=1, keepdims=True)
    e = jnp.exp(logits - m)
    z = jnp.sum(e, axis=1, keepdims=True)
    att_part = jnp.sum(e * s, axis=1, keepdims=True) / z
    lin = jnp.sum(linr_ref[...], axis=1, keepdims=True)
    o_ref[...] = jax.nn.sigmoid(bias_ref[0, 0] + lin + att_part)


def _tc_afm(e2, linr, ohi, ohj, w15, b1r, w2r, biasr):
    grid = _B // _BLK
    return pl.pallas_call(
        _tc_body,
        grid=(grid,),
        in_specs=[
            pl.BlockSpec((_D, _BLK, _F), lambda i: (0, i, 0)),
            pl.BlockSpec((_BLK, _F), lambda i: (i, 0)),
            pl.BlockSpec((_F, _P), lambda i: (0, 0)),
            pl.BlockSpec((_F, _P), lambda i: (0, 0)),
            pl.BlockSpec(memory_space=pltpu.SMEM),
            pl.BlockSpec(memory_space=pltpu.SMEM),
            pl.BlockSpec(memory_space=pltpu.SMEM),
            pl.BlockSpec(memory_space=pltpu.SMEM),
        ],
        out_specs=pl.BlockSpec((_BLK, 1), lambda i: (i, 0)),
        out_shape=jax.ShapeDtypeStruct((_B, 1), jnp.float32),
    )(e2, linr, ohi, ohj, w15, b1r, w2r, biasr)


def kernel(x, emb_table, lin_table, bias, p, W1, b1, W2):
    idx = x.astype(jnp.int32).reshape(-1)
    rows, linrows = _sc_gather_fn()(emb_table, lin_table.reshape(-1), idx)
    e2 = rows.reshape(_B, _F, _D).transpose(2, 0, 1)
    linr = linrows.reshape(_B, _F)
    w15 = jnp.concatenate([W1, p[:, None]], axis=1).T
    return _tc_afm(e2, linr, jnp.asarray(_OHI), jnp.asarray(_OHJ), w15,
                   b1.reshape(1, 4), W2.reshape(1, 4), bias.reshape(1, 1))
